# Initial kernel scaffold; baseline (speedup 1.0000x reference)
import jax
import jax.numpy as jnp
from jax.experimental import pallas as pl


def kernel(x_cat, x_cont, tables, bn_c_g, bn_c_b, W1, b1, g1, be1, W2, b2, g2, be2, W3, b3):
    raise NotImplementedError("write your pallas kernel here")


# speedup vs baseline: 7.9561x; 7.9561x over previous
import functools

import jax
import jax.numpy as jnp
from jax import lax
from jax.experimental import pallas as pl
from jax.experimental.pallas import tpu as pltpu
from jax.experimental.pallas import tpu_sc as plsc

_N_FIELDS = 26
_VOCAB = 100000
_EMB = 32
_N_CONT = 13
_BATCH = 16384
_H1, _H2 = 512, 256
_EPS = 1e-5

_ROWS = _BATCH * _N_FIELDS
_NW = 32
_ROWS_W = _ROWS // _NW
_CHUNK = 128
_NCHUNK = _ROWS_W // _CHUNK
_GROUP = 8
_NGROUP = _NCHUNK // _GROUP
_GROUP_ROWS = _GROUP * _CHUNK


def _sc_gather(table2d, idx3d):
    mesh = plsc.VectorSubcoreMesh(core_axis_name="c", subcore_axis_name="s")

    @functools.partial(
        pl.kernel,
        out_type=jax.ShapeDtypeStruct((_ROWS, _EMB), jnp.float32),
        mesh=mesh,
        scratch_types=[
            pltpu.VMEM((_NCHUNK, _CHUNK), jnp.int32),
            pltpu.VMEM((_GROUP_ROWS, _EMB), jnp.float32),
            pltpu.SemaphoreType.DMA,
        ],
        compiler_params=pltpu.CompilerParams(use_tc_tiling_on_sc=False),
    )
    def gather_kernel(table_hbm, idx_hbm, out_hbm, idx_v, rows_v, sem):
        wid = lax.axis_index("s") * 2 + lax.axis_index("c")
        base = wid * _ROWS_W
        pltpu.sync_copy(idx_hbm.at[wid], idx_v)

        def group_body(g, carry):
            copies = []
            for b in range(_GROUP):
                c = pltpu.async_copy(
                    table_hbm.at[idx_v.at[g * _GROUP + b]],
                    rows_v.at[pl.ds(b * _CHUNK, _CHUNK)],
                    sem,
                )
                copies.append(c)
            for c in copies:
                c.wait()
            pltpu.sync_copy(
                rows_v, out_hbm.at[pl.ds(base + g * _GROUP_ROWS, _GROUP_ROWS)]
            )
            return carry

        lax.fori_loop(0, _NGROUP, group_body, 0)

    return gather_kernel(table2d, idx3d)


def _mlp(emb_flat, x_cont, w1e, w1c, b1f, w2f, b2f, w3f, b3f):
    bm = 512
    grid = (_BATCH // bm,)
    d_emb = _N_FIELDS * _EMB

    def body(emb_ref, cont_ref, w1e_ref, w1c_ref, b1_ref, w2_ref, b2_ref,
             w3_ref, b3_ref, out_ref):
        h = jnp.dot(emb_ref[...], w1e_ref[...], preferred_element_type=jnp.float32)
        h = h + jnp.dot(cont_ref[...], w1c_ref[...], preferred_element_type=jnp.float32)
        h = jnp.maximum(h + b1_ref[...], 0.0)
        h = jnp.dot(h, w2_ref[...], preferred_element_type=jnp.float32) + b2_ref[...]
        h = jnp.maximum(h, 0.0)
        out_ref[...] = (
            jnp.dot(h, w3_ref[...], preferred_element_type=jnp.float32) + b3_ref[...]
        )

    const = lambda shape: pl.BlockSpec(shape, lambda i: (0, 0))
    return pl.pallas_call(
        body,
        grid=grid,
        in_specs=[
            pl.BlockSpec((bm, d_emb), lambda i: (i, 0)),
            pl.BlockSpec((bm, _N_CONT), lambda i: (i, 0)),
            const((d_emb, _H1)),
            const((_N_CONT, _H1)),
            const((1, _H1)),
            const((_H1, _H2)),
            const((1, _H2)),
            const((_H2, 1)),
            const((1, 1)),
        ],
        out_specs=pl.BlockSpec((bm, 1), lambda i: (i, 0)),
        out_shape=jax.ShapeDtypeStruct((_BATCH, 1), jnp.float32),
    )(emb_flat, x_cont, w1e, w1c, b1f, w2f, b2f, w3f, b3f)


def kernel(x_cat, x_cont, tables, bn_c_g, bn_c_b, W1, b1, g1, be1, W2, b2,
           g2, be2, W3, b3):
    s = 1.0 / jnp.sqrt(1.0 + _EPS)
    d_emb = _N_FIELDS * _EMB

    xc = jnp.clip(x_cat, 0, _VOCAB - 1)
    flat_idx = (xc + jnp.arange(_N_FIELDS, dtype=jnp.int32)[None, :] * _VOCAB)
    idx3d = flat_idx.reshape(_NW, _NCHUNK, _CHUNK)
    table2d = tables.reshape(_N_FIELDS * _VOCAB, _EMB)

    emb_flat = _sc_gather(table2d, idx3d).reshape(_BATCH, d_emb)

    w1e = W1[:d_emb]
    w1c_raw = W1[d_emb:]
    w1c = (bn_c_g * s)[:, None] * w1c_raw
    b1f = (b1 + bn_c_b @ w1c_raw)[None, :]
    w2f = (g1 * s)[:, None] * W2
    b2f = (b2 + be1 @ W2)[None, :]
    w3f = (g2 * s)[:, None] * W3
    b3f = (b3 + be2 @ W3)[None, :]

    return _mlp(emb_flat, x_cont, w1e, w1c, b1f, w2f, b2f, w3f, b3f)
